# baseline (speedup 1.0000x reference)
import functools

import numpy as np
import jax
import jax.numpy as jnp
from jax import lax
from jax.experimental import pallas as pl
from jax.experimental.pallas import tpu as pltpu
from jax.experimental.pallas import tpu_sc as plsc

_N = 10000
_E = 320000
_F = 128
_H = 4
_DH = 32
_K = 32
_CUT = 5.0
_NSPEC = 100
_AVG = 32.0
_SPHC = 32.0

_NC = 2
_NS = 16
_NW = _NC * _NS

_NPAD = 10240
_DR = 384
_DS = 512

_BE = 1280
_BN = 2000

_F32 = jnp.float32


def _pick_chunk(bpw):
    for ch in range(128, 7, -8):
        if bpw % ch == 0:
            return ch
    raise ValueError(bpw)


def _pick_group(bpw, d, cap=430_000, kord=(5, 4, 2, 1)):
    for k in kord:
        for ch in range(128, 7, -8):
            if bpw % (ch * k) == 0 and k * ch * d * 4 <= cap:
                return ch, k
    raise ValueError((bpw, d))


def _sc_params(d):
    if d % 128 == 0:
        return None
    return pltpu.CompilerParams(use_tc_tiling_on_sc=False)


def _gather_phase(table, idx, out, idx_v, rows_v, sem_g, sem_w,
                  base, bpw, ch, kk):
    g_iters = bpw // (ch * kk)

    def body(g, carry):
        off = base + g * (kk * ch)
        pltpu.sync_copy(idx.at[pl.ds(off, kk * ch)], idx_v)
        for j in range(kk):
            pltpu.async_copy(
                table.at[idx_v.at[pl.ds(j * ch, ch)]], rows_v.at[j], sem_g)
        for j in range(kk):
            pltpu.make_async_copy(
                table.at[idx_v.at[pl.ds(j * ch, ch)]], rows_v.at[j], sem_g
            ).wait()
            pltpu.async_copy(
                rows_v.at[j], out.at[pl.ds(off + j * ch, ch)], sem_w)
        for j in range(kk):
            pltpu.make_async_copy(
                rows_v.at[j], out.at[pl.ds(off + j * ch, ch)], sem_w
            ).wait()
        return carry

    lax.fori_loop(0, g_iters, body, 0)


@functools.lru_cache(maxsize=None)
def _gather2_kernel(n_rows, d1, d2, b):
    assert b % _NW == 0 and d1 % 128 == 0 and d2 % 128 == 0
    bpw = b // _NW
    ch1, k1 = _pick_group(bpw, d1, cap=310_000)
    ch2, k2 = _pick_group(bpw, d2, cap=170_000, kord=(2,))
    mesh = plsc.VectorSubcoreMesh(core_axis_name="c", subcore_axis_name="s")

    @functools.partial(
        pl.kernel,
        out_type=(jax.ShapeDtypeStruct((b, d1), _F32),
                  jax.ShapeDtypeStruct((b, d2), _F32)),
        mesh=mesh,
        scratch_types=[
            pltpu.VMEM((k1 * ch1,), jnp.int32),
            pltpu.VMEM((k1, ch1, d1), _F32),
            pltpu.VMEM((k2 * ch2,), jnp.int32),
            pltpu.VMEM((k2, ch2, d2), _F32),
            pltpu.SemaphoreType.DMA,
            pltpu.SemaphoreType.DMA,
        ],
    )
    def k(tab1, idx1, tab2, idx2, out1, out2,
          ix1, rows1, ix2, rows2, sem_g, sem_w):
        wid = lax.axis_index("s") * _NC + lax.axis_index("c")
        base = wid * bpw
        _gather_phase(tab1, idx1, out1, ix1, rows1, sem_g, sem_w,
                      base, bpw, ch1, k1)
        _gather_phase(tab2, idx2, out2, ix2, rows2, sem_g, sem_w,
                      base, bpw, ch2, k2)

    return k


@functools.lru_cache(maxsize=None)
def _scatter_kernel(d, b):
    assert b % _NW == 0 and d % 8 == 0
    bpw = b // _NW
    ch, kk = _pick_group(bpw, d, cap=110_000)
    g_iters = bpw // (ch * kk)
    rpt = _NPAD // _NS
    mesh = plsc.VectorSubcoreMesh(core_axis_name="c", subcore_axis_name="s")

    scratch = [pltpu.VMEM((ch,), jnp.int32) for _ in range(kk)]
    scratch += [
        pltpu.VMEM((kk, ch, d), _F32),
        pltpu.VMEM_SHARED((_NPAD, d), _F32),
        pltpu.SemaphoreType.DMA,
        pltpu.SemaphoreType.DMA,
    ]

    @functools.partial(
        pl.kernel,
        out_type=jax.ShapeDtypeStruct((_NC, _NPAD, d), _F32),
        mesh=mesh,
        compiler_params=_sc_params(d),
        scratch_types=scratch,
    )
    def k(msgs, idx, zeros, out, *rest):
        ixs = rest[:kk]
        msg_v, acc, sem_in, sem_sc = rest[kk:]
        c = lax.axis_index("c")
        s = lax.axis_index("s")
        wid = s * _NC + c
        base = wid * bpw
        pltpu.sync_copy(zeros.at[pl.ds(s * rpt, rpt)], acc.at[pl.ds(s * rpt, rpt)])
        plsc.subcore_barrier()

        def body(g, carry):
            off = base + g * (kk * ch)
            for j in range(kk):
                pltpu.async_copy(idx.at[pl.ds(off + j * ch, ch)], ixs[j], sem_in)
                pltpu.async_copy(msgs.at[pl.ds(off + j * ch, ch)], msg_v.at[j], sem_in)
            for j in range(kk):
                pltpu.make_async_copy(
                    idx.at[pl.ds(off + j * ch, ch)], ixs[j], sem_in).wait()
                pltpu.make_async_copy(
                    msgs.at[pl.ds(off + j * ch, ch)], msg_v.at[j], sem_in).wait()
                pltpu.sync_copy(msg_v.at[j], acc.at[ixs[j]], add=True)
            return carry

        lax.fori_loop(0, g_iters, body, 0)
        plsc.subcore_barrier()
        pltpu.sync_copy(acc.at[pl.ds(s * rpt, rpt)], out.at[c, pl.ds(s * rpt, rpt)])

    return k


def _gather2_rows(tab1, idx1, tab2, idx2):
    n_rows = tab1.shape[0]
    return _gather2_kernel(n_rows, tab1.shape[1], tab2.shape[1], idx1.shape[0])(
        tab1, idx1, tab2, idx2)


def _scatter_rows(msgs, idx):
    b, d = msgs.shape
    zeros = jnp.zeros((_NPAD, d), _F32)
    return _scatter_kernel(d, b)(msgs, idx, zeros)


def _rows_spec(nrows, ncols):
    return pl.BlockSpec((nrows, ncols), lambda i: (i, 0))


def _full_spec(shape):
    return pl.BlockSpec(shape, lambda i: tuple(0 for _ in shape))


def _silu(x):
    return x * jax.nn.sigmoid(x)


def _deg_mask():
    r = lax.broadcasted_iota(jnp.int32, (8, 2), 0)
    c = lax.broadcasted_iota(jnp.int32, (8, 2), 1)
    deg = (r >= 3).astype(jnp.int32)
    return (deg == c).astype(_F32)


def _rep_mask():
    r = lax.broadcasted_iota(jnp.int32, (2, 8), 0)
    c = lax.broadcasted_iota(jnp.int32, (2, 8), 1)
    deg = (c >= 3).astype(jnp.int32)
    return (r == deg).astype(_F32)


def _head_masks():
    r = lax.broadcasted_iota(jnp.int32, (_F, _H), 0)
    c = lax.broadcasted_iota(jnp.int32, (_F, _H), 1)
    hm = ((r // _DH) == c).astype(_F32)
    r2 = lax.broadcasted_iota(jnp.int32, (_H, _F), 0)
    c2 = lax.broadcasted_iota(jnp.int32, (_H, _F), 1)
    hmT = (r2 == (c2 // _DH)).astype(_F32)
    return hm, hmT


def _dot(a, b):
    return jnp.dot(a, b, preferred_element_type=_F32)


def _geom_body(x_ref, y_ref, z_ref, cut_ref, sh_ref, csh_ref):
    x = x_ref[...]
    y = y_ref[...]
    z = z_ref[...]
    cut = cut_ref[...]
    inv = 1.0 / (jnp.sqrt(x * x + y * y + z * z) + 1e-9)
    ux = x * inv
    uy = y * inv
    uz = z * inv
    s3 = float(np.sqrt(3.0))
    s15 = float(np.sqrt(15.0))
    s5h = float(np.sqrt(5.0) / 2.0)
    sh = jnp.concatenate(
        [
            s3 * ux, s3 * uy, s3 * uz,
            s15 * ux * uy, s15 * uy * uz,
            s5h * (3.0 * uz * uz - 1.0),
            s15 * ux * uz,
            (s15 / 2.0) * (ux * ux - uy * uy),
        ],
        axis=1,
    )
    sh_ref[...] = sh
    csh_ref[...] = sh * cut


def _geom_call(x, y, z, cut):
    grid = (_E // _BE,)
    return pl.pallas_call(
        _geom_body,
        grid=grid,
        in_specs=[_rows_spec(_BE, 1)] * 4,
        out_specs=[_rows_spec(_BE, 8), _rows_spec(_BE, 8)],
        out_shape=[
            jax.ShapeDtypeStruct((_E, 8), _F32),
            jax.ShapeDtypeStruct((_E, 8), _F32),
        ],
    )(x, y, z, cut)


def _embed_body(sp_ref, emb_ref, out_ref):
    sp = sp_ref[...]
    oh = (sp == lax.broadcasted_iota(jnp.int32, (sp.shape[0], _NSPEC), 1))
    out_ref[...] = _dot(oh.astype(_F32), emb_ref[...])


def _embed_call(sp, emb):
    grid = (_N // _BN,)
    return pl.pallas_call(
        _embed_body,
        grid=grid,
        in_specs=[_rows_spec(_BN, 1), _full_spec((_NSPEC, _F))],
        out_specs=_rows_spec(_BN, _F),
        out_shape=jax.ShapeDtypeStruct((_N, _F), _F32),
    )(sp, emb)


def _proj_body(scale, nf_ref, ca_ref, cb_ref, w_ref, b_ref,
               r_ref, s_ref, chi_ref):
    nf = nf_ref[...]
    chi = (ca_ref[...] + cb_ref[...]) * scale
    proj = _dot(nf, w_ref[...]) + b_ref[...]
    q = proj[:, 0:128]
    k = proj[:, 128:256]
    v = proj[:, 256:384]
    qg = proj[:, 384:512]
    kg = proj[:, 512:640]
    pad = jnp.zeros((nf.shape[0], 120), _F32)
    r_ref[...] = jnp.concatenate([q, qg, chi, pad], axis=1)
    s_ref[...] = jnp.concatenate([k, v, kg, chi, pad], axis=1)
    chi_ref[...] = chi


def _proj_call(scale, nf, chi_parts, wcat, bcat):
    grid = (_N // _BN,)
    return pl.pallas_call(
        functools.partial(_proj_body, scale),
        grid=grid,
        in_specs=[
            _rows_spec(_BN, _F),
            _rows_spec(_BN, 8),
            _rows_spec(_BN, 8),
            _full_spec((_F, 5 * _F)),
            _full_spec((1, 5 * _F)),
        ],
        out_specs=[
            _rows_spec(_BN, _DR),
            _rows_spec(_BN, _DS),
            _rows_spec(_BN, 8),
        ],
        out_shape=[
            jax.ShapeDtypeStruct((_N, _DR), _F32),
            jax.ShapeDtypeStruct((_N, _DS), _F32),
            jax.ShapeDtypeStruct((_N, 8), _F32),
        ],
    )(nf, *chi_parts, wcat, bcat)


def _edge_body(d_ref, cut_ref, sh_ref, q_ref, qg_ref, chir_ref,
               k_ref, v_ref, kg_ref, chis_ref,
               w1r_ref, b1r_ref, w2r_ref, b2r_ref,
               w1s_ref, b1s_ref, w2s_ref, b2s_ref,
               g1r_ref, gb1r_ref, g2r_ref, gb2r_ref,
               g1s_ref, gb1s_ref, g2s_ref, gb2s_ref,
               msg_ref, msgc_ref):
    d = d_ref[...]
    cut = cut_ref[...]
    sh = sh_ref[...]
    q = q_ref[...]
    qg = qg_ref[...]
    chir = chir_ref[:, 0:8]
    k = k_ref[...]
    v = v_ref[...]
    kg = kg_ref[...]
    chis = chis_ref[:, 0:8]

    mu0 = float(np.exp(-_CUT))
    mu = mu0 + lax.broadcasted_iota(jnp.int32, (1, _K), 1).astype(_F32) * (
        (1.0 - mu0) / (_K - 1))
    beta = float((2.0 / _K * (1.0 - np.exp(-_CUT))) ** (-2))
    t = jnp.exp(-d) - mu
    rbf = jnp.exp(-beta * t * t)

    chi_ij = chis - chir
    chi_sc = _dot(chi_ij * chi_ij, _deg_mask())

    def mlp2(x, w1, b1, w2, b2):
        h = _silu(_dot(x, w1) + b1)
        return _dot(h, w2) + b2

    w = mlp2(rbf, w1r_ref[...], b1r_ref[...], w2r_ref[...], b2r_ref[...]) + \
        mlp2(chi_sc, w1s_ref[...], b1s_ref[...], w2s_ref[...], b2s_ref[...])
    wg = mlp2(rbf, g1r_ref[...], gb1r_ref[...], g2r_ref[...], gb2r_ref[...]) + \
        mlp2(chi_sc, g1s_ref[...], gb1s_ref[...], g2s_ref[...], gb2s_ref[...])

    hm, hmT = _head_masks()
    alpha = _dot(q * w * k, hm) * (cut * (1.0 / float(np.sqrt(_DH))))
    ag = jnp.sum(qg * kg, axis=1, keepdims=True) * (1.0 / float(np.sqrt(_F)))
    coeff = wg * (ag * cut)
    msg_ref[...] = _dot(alpha, hmT) * v
    msgc_ref[...] = _dot(coeff, _rep_mask()) * sh


def _col_spec(nrows, ncols, col0):
    return pl.BlockSpec((nrows, ncols), lambda i, _c=col0: (i, _c))


def _edge_call(d, cut, sh, gr, gs, wts, ne):
    grid = (ne // _BE,)
    w_specs = [_full_spec(w.shape) for w in wts]
    return pl.pallas_call(
        _edge_body,
        grid=grid,
        in_specs=[
            _rows_spec(_BE, 1),
            _rows_spec(_BE, 1),
            _rows_spec(_BE, 8),
            _col_spec(_BE, 128, 0),
            _col_spec(_BE, 128, 1),
            _col_spec(_BE, 128, 2),
            _col_spec(_BE, 128, 0),
            _col_spec(_BE, 128, 1),
            _col_spec(_BE, 128, 2),
            _col_spec(_BE, 128, 3),
        ] + w_specs,
        out_specs=[_rows_spec(_BE, _F), _rows_spec(_BE, 8)],
        out_shape=[
            jax.ShapeDtypeStruct((ne, _F), _F32),
            jax.ShapeDtypeStruct((ne, 8), _F32),
        ],
    )(d, cut, sh, gr, gr, gr, gs, gs, gs, gs, *wts)


def _update_body(nf_ref, chi_ref, a0_ref, a1_ref, c0_ref, c1_ref,
                 w_ref, b_ref, nfo_ref, chio_ref):
    nf1 = nf_ref[...] + (a0_ref[...] + a1_ref[...]) * (1.0 / _AVG)
    chi1 = chi_ref[...] + (c0_ref[...] + c1_ref[...]) * (1.0 / _AVG)
    dn = _dot(chi1 * chi1, _deg_mask())
    feat = jnp.concatenate([nf1, dn], axis=1)
    o = _silu(_dot(feat, w_ref[...]) + b_ref[...])
    nfo_ref[...] = nf1 + o[:, 0:128]
    g = _dot(o[:, 128:130], _rep_mask())
    chio_ref[...] = chi1 + chi1 * g


def _update_call(nf, chi, a_parts, c_parts, wi, bi):
    grid = (_N // _BN,)
    return pl.pallas_call(
        _update_body,
        grid=grid,
        in_specs=[
            _rows_spec(_BN, _F),
            _rows_spec(_BN, 8),
            _rows_spec(_BN, _F),
            _rows_spec(_BN, _F),
            _rows_spec(_BN, 8),
            _rows_spec(_BN, 8),
            _full_spec((_F + 2, _F + 2)),
            _full_spec((1, _F + 2)),
        ],
        out_specs=[_rows_spec(_BN, _F), _rows_spec(_BN, 8)],
        out_shape=[
            jax.ShapeDtypeStruct((_N, _F), _F32),
            jax.ShapeDtypeStruct((_N, 8), _F32),
        ],
    )(nf, chi, *a_parts, *c_parts, wi, bi)


def _head_body(nf_ref, w1_ref, b1_ref, w2_ref, b2_ref, out_ref):
    h = _silu(_dot(nf_ref[...], w1_ref[...]) + b1_ref[...])
    out_ref[...] = _dot(h, w2_ref[...]) + b2_ref[...]


def _head_call(nf, w1, b1, w2, b2):
    grid = (_N // _BN,)
    return pl.pallas_call(
        _head_body,
        grid=grid,
        in_specs=[
            _rows_spec(_BN, _F),
            _full_spec((_F, _F)),
            _full_spec((1, _F)),
            _full_spec((_F, 1)),
            _full_spec((1, 1)),
        ],
        out_specs=_rows_spec(_BN, 1),
        out_shape=jax.ShapeDtypeStruct((_N, 1), _F32),
    )(nf, w1, b1, w2, b2)


def kernel(edge_vectors, distances, cutoffs, node_species, senders, receivers, params):
    x = edge_vectors[:, 0:1]
    y = edge_vectors[:, 1:2]
    z = edge_vectors[:, 2:3]
    d = distances.reshape(_E, 1)
    cut = cutoffs.reshape(_E, 1)
    sp = node_species.reshape(_N, 1).astype(jnp.int32)
    snd = senders.astype(jnp.int32)
    rcv = receivers.astype(jnp.int32)

    sh, csh = _geom_call(x, y, z, cut)
    p = _scatter_rows(csh, rcv)
    chi_parts = [p[0, :_N], p[1, :_N]]
    nf = _embed_call(sp, params['embed'])

    zeros8 = jnp.zeros((_N, 8), _F32)
    scale = 1.0 / _SPHC

    for lp in params['layers']:
        wcat = jnp.concatenate(
            [lp['fb_q'][0], lp['fb_k'][0], lp['fb_v'][0], lp['gb_q'][0], lp['gb_k'][0]],
            axis=1,
        )
        bcat = jnp.concatenate(
            [lp['fb_q'][1], lp['fb_k'][1], lp['fb_v'][1], lp['gb_q'][1], lp['gb_k'][1]],
        ).reshape(1, 5 * _F)
        r_tab, s_tab, chi = _proj_call(scale, nf, chi_parts, wcat, bcat)
        wts = [
            lp['fb_rad'][0][0], lp['fb_rad'][0][1].reshape(1, -1),
            lp['fb_rad'][1][0], lp['fb_rad'][1][1].reshape(1, -1),
            lp['fb_sph'][0][0], lp['fb_sph'][0][1].reshape(1, -1),
            lp['fb_sph'][1][0], lp['fb_sph'][1][1].reshape(1, -1),
            lp['gb_rad'][0][0], lp['gb_rad'][0][1].reshape(1, -1),
            lp['gb_rad'][1][0], lp['gb_rad'][1][1].reshape(1, -1),
            lp['gb_sph'][0][0], lp['gb_sph'][0][1].reshape(1, -1),
            lp['gb_sph'][1][0], lp['gb_sph'][1][1].reshape(1, -1),
        ]
        g_r, g_s = _gather2_rows(r_tab, rcv, s_tab, snd)
        msg_nf, msg_chi = _edge_call(d, cut, sh, g_r, g_s, wts, _E)
        acc = _scatter_rows(msg_nf, rcv)
        accc = _scatter_rows(msg_chi, rcv)
        nf, chi_next = _update_call(nf, chi,
                                    [acc[0, :_N], acc[1, :_N]],
                                    [accc[0, :_N], accc[1, :_N]],
                                    lp['inter'][0], lp['inter'][1].reshape(1, -1))
        chi_parts, scale = [chi_next, zeros8], 1.0

    out = _head_call(nf, params['out1'][0], params['out1'][1].reshape(1, -1),
                     params['out2'][0], params['out2'][1].reshape(1, -1))
    return out.reshape(_N)


# speedup vs baseline: 1.0276x; 1.0276x over previous
import functools

import numpy as np
import jax
import jax.numpy as jnp
from jax import lax
from jax.experimental import pallas as pl
from jax.experimental.pallas import tpu as pltpu
from jax.experimental.pallas import tpu_sc as plsc

_N = 10000
_E = 320000
_F = 128
_H = 4
_DH = 32
_K = 32
_CUT = 5.0
_NSPEC = 100
_AVG = 32.0
_SPHC = 32.0

_NC = 2
_NS = 16
_NW = _NC * _NS

_NPAD = 10240
_DR = 384
_DS = 512

_BE = 1280
_BN = 2000

_F32 = jnp.float32


def _pick_chunk(bpw):
    for ch in range(128, 7, -8):
        if bpw % ch == 0:
            return ch
    raise ValueError(bpw)


def _pick_group(bpw, d, cap=430_000, kord=(5, 4, 2, 1)):
    for k in kord:
        for ch in range(128, 7, -8):
            if bpw % (ch * k) == 0 and k * ch * d * 4 <= cap:
                return ch, k
    raise ValueError((bpw, d))


def _sc_params(d):
    if d % 128 == 0:
        return None
    return pltpu.CompilerParams(use_tc_tiling_on_sc=False)


def _gather_phase(table, idx, out, idx_v, rows_v, sem_g, sem_w,
                  base, bpw, ch, kk):
    g_iters = bpw // (ch * kk)

    def body(g, carry):
        off = base + g * (kk * ch)
        pltpu.sync_copy(idx.at[pl.ds(off, kk * ch)], idx_v)
        for j in range(kk):
            pltpu.async_copy(
                table.at[idx_v.at[pl.ds(j * ch, ch)]], rows_v.at[j], sem_g)
        for j in range(kk):
            pltpu.make_async_copy(
                table.at[idx_v.at[pl.ds(j * ch, ch)]], rows_v.at[j], sem_g
            ).wait()
            pltpu.async_copy(
                rows_v.at[j], out.at[pl.ds(off + j * ch, ch)], sem_w)
        for j in range(kk):
            pltpu.make_async_copy(
                rows_v.at[j], out.at[pl.ds(off + j * ch, ch)], sem_w
            ).wait()
        return carry

    lax.fori_loop(0, g_iters, body, 0)


@functools.lru_cache(maxsize=None)
def _gather_kernel(n_rows, d, b):
    assert b % _NW == 0 and d % 8 == 0
    bpw = b // _NW
    ch, kk = _pick_group(bpw, d)
    mesh = plsc.VectorSubcoreMesh(core_axis_name="c", subcore_axis_name="s")

    @functools.partial(
        pl.kernel,
        out_type=jax.ShapeDtypeStruct((b, d), _F32),
        mesh=mesh,
        compiler_params=_sc_params(d),
        scratch_types=[
            pltpu.VMEM((kk * ch,), jnp.int32),
            pltpu.VMEM((kk, ch, d), _F32),
            pltpu.SemaphoreType.DMA,
            pltpu.SemaphoreType.DMA,
        ],
    )
    def k(table, idx, out, idx_v, rows_v, sem_g, sem_w):
        wid = lax.axis_index("s") * _NC + lax.axis_index("c")
        base = wid * bpw
        _gather_phase(table, idx, out, idx_v, rows_v, sem_g, sem_w,
                      base, bpw, ch, kk)

    return k


@functools.lru_cache(maxsize=None)
def _scatter_kernel(d, b):
    assert b % _NW == 0 and d % 8 == 0
    bpw = b // _NW
    ch, kk = _pick_group(bpw, d, cap=110_000)
    g_iters = bpw // (ch * kk)
    rpt = _NPAD // _NS
    mesh = plsc.VectorSubcoreMesh(core_axis_name="c", subcore_axis_name="s")

    scratch = [pltpu.VMEM((ch,), jnp.int32) for _ in range(kk)]
    scratch += [
        pltpu.VMEM((kk, ch, d), _F32),
        pltpu.VMEM_SHARED((_NPAD, d), _F32),
        pltpu.SemaphoreType.DMA,
        pltpu.SemaphoreType.DMA,
    ]

    @functools.partial(
        pl.kernel,
        out_type=jax.ShapeDtypeStruct((_NC, _NPAD, d), _F32),
        mesh=mesh,
        compiler_params=_sc_params(d),
        scratch_types=scratch,
    )
    def k(msgs, idx, zeros, out, *rest):
        ixs = rest[:kk]
        msg_v, acc, sem_in, sem_sc = rest[kk:]
        c = lax.axis_index("c")
        s = lax.axis_index("s")
        wid = s * _NC + c
        base = wid * bpw
        pltpu.sync_copy(zeros.at[pl.ds(s * rpt, rpt)], acc.at[pl.ds(s * rpt, rpt)])
        plsc.subcore_barrier()

        def body(g, carry):
            off = base + g * (kk * ch)
            for j in range(kk):
                pltpu.async_copy(idx.at[pl.ds(off + j * ch, ch)], ixs[j], sem_in)
                pltpu.async_copy(msgs.at[pl.ds(off + j * ch, ch)], msg_v.at[j], sem_in)
            for j in range(kk):
                pltpu.make_async_copy(
                    idx.at[pl.ds(off + j * ch, ch)], ixs[j], sem_in).wait()
                pltpu.make_async_copy(
                    msgs.at[pl.ds(off + j * ch, ch)], msg_v.at[j], sem_in).wait()
                pltpu.sync_copy(msg_v.at[j], acc.at[ixs[j]], add=True)
            return carry

        lax.fori_loop(0, g_iters, body, 0)
        plsc.subcore_barrier()
        pltpu.sync_copy(acc.at[pl.ds(s * rpt, rpt)], out.at[c, pl.ds(s * rpt, rpt)])

    return k


def _gather_rows(table, idx):
    n_rows, d = table.shape
    return _gather_kernel(n_rows, d, idx.shape[0])(table, idx)


def _scatter_rows(msgs, idx):
    b, d = msgs.shape
    zeros = jnp.zeros((_NPAD, d), _F32)
    return _scatter_kernel(d, b)(msgs, idx, zeros)


def _rows_spec(nrows, ncols):
    return pl.BlockSpec((nrows, ncols), lambda i: (i, 0))


def _full_spec(shape):
    return pl.BlockSpec(shape, lambda i: tuple(0 for _ in shape))


def _silu(x):
    return x * jax.nn.sigmoid(x)


def _deg_mask():
    r = lax.broadcasted_iota(jnp.int32, (8, 2), 0)
    c = lax.broadcasted_iota(jnp.int32, (8, 2), 1)
    deg = (r >= 3).astype(jnp.int32)
    return (deg == c).astype(_F32)


def _rep_mask():
    r = lax.broadcasted_iota(jnp.int32, (2, 8), 0)
    c = lax.broadcasted_iota(jnp.int32, (2, 8), 1)
    deg = (c >= 3).astype(jnp.int32)
    return (r == deg).astype(_F32)


def _head_masks():
    r = lax.broadcasted_iota(jnp.int32, (_F, _H), 0)
    c = lax.broadcasted_iota(jnp.int32, (_F, _H), 1)
    hm = ((r // _DH) == c).astype(_F32)
    r2 = lax.broadcasted_iota(jnp.int32, (_H, _F), 0)
    c2 = lax.broadcasted_iota(jnp.int32, (_H, _F), 1)
    hmT = (r2 == (c2 // _DH)).astype(_F32)
    return hm, hmT


def _dot(a, b):
    return jnp.dot(a, b, preferred_element_type=_F32)


def _geom_body(x_ref, y_ref, z_ref, cut_ref, sh_ref, csh_ref):
    x = x_ref[...]
    y = y_ref[...]
    z = z_ref[...]
    cut = cut_ref[...]
    inv = 1.0 / (jnp.sqrt(x * x + y * y + z * z) + 1e-9)
    ux = x * inv
    uy = y * inv
    uz = z * inv
    s3 = float(np.sqrt(3.0))
    s15 = float(np.sqrt(15.0))
    s5h = float(np.sqrt(5.0) / 2.0)
    sh = jnp.concatenate(
        [
            s3 * ux, s3 * uy, s3 * uz,
            s15 * ux * uy, s15 * uy * uz,
            s5h * (3.0 * uz * uz - 1.0),
            s15 * ux * uz,
            (s15 / 2.0) * (ux * ux - uy * uy),
        ],
        axis=1,
    )
    sh_ref[...] = sh
    csh_ref[...] = sh * cut


def _geom_call(x, y, z, cut):
    grid = (_E // _BE,)
    return pl.pallas_call(
        _geom_body,
        grid=grid,
        in_specs=[_rows_spec(_BE, 1)] * 4,
        out_specs=[_rows_spec(_BE, 8), _rows_spec(_BE, 8)],
        out_shape=[
            jax.ShapeDtypeStruct((_E, 8), _F32),
            jax.ShapeDtypeStruct((_E, 8), _F32),
        ],
    )(x, y, z, cut)


def _embed_body(sp_ref, emb_ref, out_ref):
    sp = sp_ref[...]
    oh = (sp == lax.broadcasted_iota(jnp.int32, (sp.shape[0], _NSPEC), 1))
    out_ref[...] = _dot(oh.astype(_F32), emb_ref[...])


def _embed_call(sp, emb):
    grid = (_N // _BN,)
    return pl.pallas_call(
        _embed_body,
        grid=grid,
        in_specs=[_rows_spec(_BN, 1), _full_spec((_NSPEC, _F))],
        out_specs=_rows_spec(_BN, _F),
        out_shape=jax.ShapeDtypeStruct((_N, _F), _F32),
    )(sp, emb)


def _proj_body(scale, nf_ref, ca_ref, cb_ref, w_ref, b_ref,
               r_ref, s_ref, chi_ref):
    nf = nf_ref[...]
    chi = (ca_ref[...] + cb_ref[...]) * scale
    proj = _dot(nf, w_ref[...]) + b_ref[...]
    q = proj[:, 0:128]
    k = proj[:, 128:256]
    v = proj[:, 256:384]
    qg = proj[:, 384:512]
    kg = proj[:, 512:640]
    pad = jnp.zeros((nf.shape[0], 120), _F32)
    r_ref[...] = jnp.concatenate([q, qg, chi, pad], axis=1)
    s_ref[...] = jnp.concatenate([k, v, kg, chi, pad], axis=1)
    chi_ref[...] = chi


def _proj_call(scale, nf, chi_parts, wcat, bcat):
    grid = (_N // _BN,)
    return pl.pallas_call(
        functools.partial(_proj_body, scale),
        grid=grid,
        in_specs=[
            _rows_spec(_BN, _F),
            _rows_spec(_BN, 8),
            _rows_spec(_BN, 8),
            _full_spec((_F, 5 * _F)),
            _full_spec((1, 5 * _F)),
        ],
        out_specs=[
            _rows_spec(_BN, _DR),
            _rows_spec(_BN, _DS),
            _rows_spec(_BN, 8),
        ],
        out_shape=[
            jax.ShapeDtypeStruct((_N, _DR), _F32),
            jax.ShapeDtypeStruct((_N, _DS), _F32),
            jax.ShapeDtypeStruct((_N, 8), _F32),
        ],
    )(nf, *chi_parts, wcat, bcat)


def _edge_body(d_ref, cut_ref, sh_ref, q_ref, qg_ref, chir_ref,
               k_ref, v_ref, kg_ref, chis_ref,
               w1r_ref, b1r_ref, w2r_ref, b2r_ref,
               w1s_ref, b1s_ref, w2s_ref, b2s_ref,
               g1r_ref, gb1r_ref, g2r_ref, gb2r_ref,
               g1s_ref, gb1s_ref, g2s_ref, gb2s_ref,
               msg_ref, msgc_ref):
    d = d_ref[...]
    cut = cut_ref[...]
    sh = sh_ref[...]
    q = q_ref[...]
    qg = qg_ref[...]
    chir = chir_ref[:, 0:8]
    k = k_ref[...]
    v = v_ref[...]
    kg = kg_ref[...]
    chis = chis_ref[:, 0:8]

    mu0 = float(np.exp(-_CUT))
    mu = mu0 + lax.broadcasted_iota(jnp.int32, (1, _K), 1).astype(_F32) * (
        (1.0 - mu0) / (_K - 1))
    beta = float((2.0 / _K * (1.0 - np.exp(-_CUT))) ** (-2))
    t = jnp.exp(-d) - mu
    rbf = jnp.exp(-beta * t * t)

    chi_ij = chis - chir
    chi_sc = _dot(chi_ij * chi_ij, _deg_mask())

    def mlp2(x, w1, b1, w2, b2):
        h = _silu(_dot(x, w1) + b1)
        return _dot(h, w2) + b2

    w = mlp2(rbf, w1r_ref[...], b1r_ref[...], w2r_ref[...], b2r_ref[...]) + \
        mlp2(chi_sc, w1s_ref[...], b1s_ref[...], w2s_ref[...], b2s_ref[...])
    wg = mlp2(rbf, g1r_ref[...], gb1r_ref[...], g2r_ref[...], gb2r_ref[...]) + \
        mlp2(chi_sc, g1s_ref[...], gb1s_ref[...], g2s_ref[...], gb2s_ref[...])

    hm, hmT = _head_masks()
    alpha = _dot(q * w * k, hm) * (cut * (1.0 / float(np.sqrt(_DH))))
    ag = jnp.sum(qg * kg, axis=1, keepdims=True) * (1.0 / float(np.sqrt(_F)))
    coeff = wg * (ag * cut)
    msg_ref[...] = _dot(alpha, hmT) * v
    msgc_ref[...] = _dot(coeff, _rep_mask()) * sh


def _col_spec(nrows, ncols, col0):
    return pl.BlockSpec((nrows, ncols), lambda i, _c=col0: (i, _c))


def _edge_call(d, cut, sh, gr, gs, wts, ne):
    grid = (ne // _BE,)
    w_specs = [_full_spec(w.shape) for w in wts]
    return pl.pallas_call(
        _edge_body,
        grid=grid,
        in_specs=[
            _rows_spec(_BE, 1),
            _rows_spec(_BE, 1),
            _rows_spec(_BE, 8),
            _col_spec(_BE, 128, 0),
            _col_spec(_BE, 128, 1),
            _col_spec(_BE, 128, 2),
            _col_spec(_BE, 128, 0),
            _col_spec(_BE, 128, 1),
            _col_spec(_BE, 128, 2),
            _col_spec(_BE, 128, 3),
        ] + w_specs,
        out_specs=[_rows_spec(_BE, _F), _rows_spec(_BE, 8)],
        out_shape=[
            jax.ShapeDtypeStruct((ne, _F), _F32),
            jax.ShapeDtypeStruct((ne, 8), _F32),
        ],
    )(d, cut, sh, gr, gr, gr, gs, gs, gs, gs, *wts)


def _update_body(nf_ref, chi_ref, a0_ref, a1_ref, c0_ref, c1_ref,
                 w_ref, b_ref, nfo_ref, chio_ref):
    nf1 = nf_ref[...] + (a0_ref[...] + a1_ref[...]) * (1.0 / _AVG)
    chi1 = chi_ref[...] + (c0_ref[...] + c1_ref[...]) * (1.0 / _AVG)
    dn = _dot(chi1 * chi1, _deg_mask())
    feat = jnp.concatenate([nf1, dn], axis=1)
    o = _silu(_dot(feat, w_ref[...]) + b_ref[...])
    nfo_ref[...] = nf1 + o[:, 0:128]
    g = _dot(o[:, 128:130], _rep_mask())
    chio_ref[...] = chi1 + chi1 * g


def _update_call(nf, chi, a_parts, c_parts, wi, bi):
    grid = (_N // _BN,)
    return pl.pallas_call(
        _update_body,
        grid=grid,
        in_specs=[
            _rows_spec(_BN, _F),
            _rows_spec(_BN, 8),
            _rows_spec(_BN, _F),
            _rows_spec(_BN, _F),
            _rows_spec(_BN, 8),
            _rows_spec(_BN, 8),
            _full_spec((_F + 2, _F + 2)),
            _full_spec((1, _F + 2)),
        ],
        out_specs=[_rows_spec(_BN, _F), _rows_spec(_BN, 8)],
        out_shape=[
            jax.ShapeDtypeStruct((_N, _F), _F32),
            jax.ShapeDtypeStruct((_N, 8), _F32),
        ],
    )(nf, chi, *a_parts, *c_parts, wi, bi)


def _head_body(nf_ref, w1_ref, b1_ref, w2_ref, b2_ref, out_ref):
    h = _silu(_dot(nf_ref[...], w1_ref[...]) + b1_ref[...])
    out_ref[...] = _dot(h, w2_ref[...]) + b2_ref[...]


def _head_call(nf, w1, b1, w2, b2):
    grid = (_N // _BN,)
    return pl.pallas_call(
        _head_body,
        grid=grid,
        in_specs=[
            _rows_spec(_BN, _F),
            _full_spec((_F, _F)),
            _full_spec((1, _F)),
            _full_spec((_F, 1)),
            _full_spec((1, 1)),
        ],
        out_specs=_rows_spec(_BN, 1),
        out_shape=jax.ShapeDtypeStruct((_N, 1), _F32),
    )(nf, w1, b1, w2, b2)


def kernel(edge_vectors, distances, cutoffs, node_species, senders, receivers, params):
    x = edge_vectors[:, 0:1]
    y = edge_vectors[:, 1:2]
    z = edge_vectors[:, 2:3]
    d = distances.reshape(_E, 1)
    cut = cutoffs.reshape(_E, 1)
    sp = node_species.reshape(_N, 1).astype(jnp.int32)
    snd = senders.astype(jnp.int32)
    rcv = receivers.astype(jnp.int32)

    sh, csh = _geom_call(x, y, z, cut)
    p = _scatter_rows(csh, rcv)
    chi_parts = [p[0, :_N], p[1, :_N]]
    nf = _embed_call(sp, params['embed'])

    zeros8 = jnp.zeros((_N, 8), _F32)
    scale = 1.0 / _SPHC

    for lp in params['layers']:
        wcat = jnp.concatenate(
            [lp['fb_q'][0], lp['fb_k'][0], lp['fb_v'][0], lp['gb_q'][0], lp['gb_k'][0]],
            axis=1,
        )
        bcat = jnp.concatenate(
            [lp['fb_q'][1], lp['fb_k'][1], lp['fb_v'][1], lp['gb_q'][1], lp['gb_k'][1]],
        ).reshape(1, 5 * _F)
        r_tab, s_tab, chi = _proj_call(scale, nf, chi_parts, wcat, bcat)
        wts = [
            lp['fb_rad'][0][0], lp['fb_rad'][0][1].reshape(1, -1),
            lp['fb_rad'][1][0], lp['fb_rad'][1][1].reshape(1, -1),
            lp['fb_sph'][0][0], lp['fb_sph'][0][1].reshape(1, -1),
            lp['fb_sph'][1][0], lp['fb_sph'][1][1].reshape(1, -1),
            lp['gb_rad'][0][0], lp['gb_rad'][0][1].reshape(1, -1),
            lp['gb_rad'][1][0], lp['gb_rad'][1][1].reshape(1, -1),
            lp['gb_sph'][0][0], lp['gb_sph'][0][1].reshape(1, -1),
            lp['gb_sph'][1][0], lp['gb_sph'][1][1].reshape(1, -1),
        ]
        g_r = _gather_rows(r_tab, rcv)
        g_s = _gather_rows(s_tab, snd)
        msg_nf, msg_chi = _edge_call(d, cut, sh, g_r, g_s, wts, _E)
        acc = _scatter_rows(msg_nf, rcv)
        accc = _scatter_rows(msg_chi, rcv)
        nf, chi_next = _update_call(nf, chi,
                                    [acc[0, :_N], acc[1, :_N]],
                                    [accc[0, :_N], accc[1, :_N]],
                                    lp['inter'][0], lp['inter'][1].reshape(1, -1))
        chi_parts, scale = [chi_next, zeros8], 1.0

    out = _head_call(nf, params['out1'][0], params['out1'][1].reshape(1, -1),
                     params['out2'][0], params['out2'][1].reshape(1, -1))
    return out.reshape(_N)


# speedup vs baseline: 1.0587x; 1.0303x over previous
import functools

import numpy as np
import jax
import jax.numpy as jnp
from jax import lax
from jax.experimental import pallas as pl
from jax.experimental.pallas import tpu as pltpu
from jax.experimental.pallas import tpu_sc as plsc

_N = 10000
_E = 320000
_F = 128
_H = 4
_DH = 32
_K = 32
_CUT = 5.0
_NSPEC = 100
_AVG = 32.0
_SPHC = 32.0

_NC = 2
_NS = 16
_NW = _NC * _NS

_NPAD = 10240
_DR = 384
_DS = 512

_BE = 2560
_BN = 2000

_F32 = jnp.float32


def _pick_chunk(bpw):
    for ch in range(128, 7, -8):
        if bpw % ch == 0:
            return ch
    raise ValueError(bpw)


def _pick_group(bpw, d, cap=430_000, kord=(5, 4, 2, 1)):
    for k in kord:
        for ch in range(128, 7, -8):
            if bpw % (ch * k) == 0 and k * ch * d * 4 <= cap:
                return ch, k
    raise ValueError((bpw, d))


def _sc_params(d):
    if d % 128 == 0:
        return None
    return pltpu.CompilerParams(use_tc_tiling_on_sc=False)


def _gather_phase(table, idx, out, idx_v, rows_v, sem_g, sem_w,
                  base, bpw, ch, kk):
    g_iters = bpw // (ch * kk)

    def body(g, carry):
        off = base + g * (kk * ch)
        pltpu.sync_copy(idx.at[pl.ds(off, kk * ch)], idx_v)
        for j in range(kk):
            pltpu.async_copy(
                table.at[idx_v.at[pl.ds(j * ch, ch)]], rows_v.at[j], sem_g)
        for j in range(kk):
            pltpu.make_async_copy(
                table.at[idx_v.at[pl.ds(j * ch, ch)]], rows_v.at[j], sem_g
            ).wait()
            pltpu.async_copy(
                rows_v.at[j], out.at[pl.ds(off + j * ch, ch)], sem_w)
        for j in range(kk):
            pltpu.make_async_copy(
                rows_v.at[j], out.at[pl.ds(off + j * ch, ch)], sem_w
            ).wait()
        return carry

    lax.fori_loop(0, g_iters, body, 0)


@functools.lru_cache(maxsize=None)
def _gather_kernel(n_rows, d, b):
    assert b % _NW == 0 and d % 8 == 0
    bpw = b // _NW
    ch, kk = _pick_group(bpw, d)
    mesh = plsc.VectorSubcoreMesh(core_axis_name="c", subcore_axis_name="s")

    @functools.partial(
        pl.kernel,
        out_type=jax.ShapeDtypeStruct((b, d), _F32),
        mesh=mesh,
        compiler_params=_sc_params(d),
        scratch_types=[
            pltpu.VMEM((kk * ch,), jnp.int32),
            pltpu.VMEM((kk, ch, d), _F32),
            pltpu.SemaphoreType.DMA,
            pltpu.SemaphoreType.DMA,
        ],
    )
    def k(table, idx, out, idx_v, rows_v, sem_g, sem_w):
        wid = lax.axis_index("s") * _NC + lax.axis_index("c")
        base = wid * bpw
        _gather_phase(table, idx, out, idx_v, rows_v, sem_g, sem_w,
                      base, bpw, ch, kk)

    return k


@functools.lru_cache(maxsize=None)
def _scatter_kernel(d, b):
    assert b % _NW == 0 and d % 8 == 0
    bpw = b // _NW
    ch, kk = _pick_group(bpw, d, cap=110_000)
    g_iters = bpw // (ch * kk)
    rpt = _NPAD // _NS
    mesh = plsc.VectorSubcoreMesh(core_axis_name="c", subcore_axis_name="s")

    scratch = [pltpu.VMEM((ch,), jnp.int32) for _ in range(kk)]
    scratch += [
        pltpu.VMEM((kk, ch, d), _F32),
        pltpu.VMEM_SHARED((_NPAD, d), _F32),
        pltpu.SemaphoreType.DMA,
        pltpu.SemaphoreType.DMA,
    ]

    @functools.partial(
        pl.kernel,
        out_type=jax.ShapeDtypeStruct((_NC, _NPAD, d), _F32),
        mesh=mesh,
        compiler_params=_sc_params(d),
        scratch_types=scratch,
    )
    def k(msgs, idx, zeros, out, *rest):
        ixs = rest[:kk]
        msg_v, acc, sem_in, sem_sc = rest[kk:]
        c = lax.axis_index("c")
        s = lax.axis_index("s")
        wid = s * _NC + c
        base = wid * bpw
        pltpu.sync_copy(zeros.at[pl.ds(s * rpt, rpt)], acc.at[pl.ds(s * rpt, rpt)])
        plsc.subcore_barrier()

        def body(g, carry):
            off = base + g * (kk * ch)
            for j in range(kk):
                pltpu.async_copy(idx.at[pl.ds(off + j * ch, ch)], ixs[j], sem_in)
                pltpu.async_copy(msgs.at[pl.ds(off + j * ch, ch)], msg_v.at[j], sem_in)
            for j in range(kk):
                pltpu.make_async_copy(
                    idx.at[pl.ds(off + j * ch, ch)], ixs[j], sem_in).wait()
                pltpu.make_async_copy(
                    msgs.at[pl.ds(off + j * ch, ch)], msg_v.at[j], sem_in).wait()
                pltpu.sync_copy(msg_v.at[j], acc.at[ixs[j]], add=True)
            return carry

        lax.fori_loop(0, g_iters, body, 0)
        plsc.subcore_barrier()
        pltpu.sync_copy(acc.at[pl.ds(s * rpt, rpt)], out.at[c, pl.ds(s * rpt, rpt)])

    return k


def _gather_rows(table, idx):
    n_rows, d = table.shape
    return _gather_kernel(n_rows, d, idx.shape[0])(table, idx)


def _scatter_rows(msgs, idx):
    b, d = msgs.shape
    zeros = jnp.zeros((_NPAD, d), _F32)
    return _scatter_kernel(d, b)(msgs, idx, zeros)


def _rows_spec(nrows, ncols):
    return pl.BlockSpec((nrows, ncols), lambda i: (i, 0))


def _full_spec(shape):
    return pl.BlockSpec(shape, lambda i: tuple(0 for _ in shape))


def _silu(x):
    return x * jax.nn.sigmoid(x)


def _deg_mask():
    r = lax.broadcasted_iota(jnp.int32, (8, 2), 0)
    c = lax.broadcasted_iota(jnp.int32, (8, 2), 1)
    deg = (r >= 3).astype(jnp.int32)
    return (deg == c).astype(_F32)


def _rep_mask():
    r = lax.broadcasted_iota(jnp.int32, (2, 8), 0)
    c = lax.broadcasted_iota(jnp.int32, (2, 8), 1)
    deg = (c >= 3).astype(jnp.int32)
    return (r == deg).astype(_F32)


def _head_masks():
    r = lax.broadcasted_iota(jnp.int32, (_F, _H), 0)
    c = lax.broadcasted_iota(jnp.int32, (_F, _H), 1)
    hm = ((r // _DH) == c).astype(_F32)
    r2 = lax.broadcasted_iota(jnp.int32, (_H, _F), 0)
    c2 = lax.broadcasted_iota(jnp.int32, (_H, _F), 1)
    hmT = (r2 == (c2 // _DH)).astype(_F32)
    return hm, hmT


def _dot(a, b):
    return jnp.dot(a, b, preferred_element_type=_F32)


def _geom_body(x_ref, y_ref, z_ref, cut_ref, sh_ref, csh_ref):
    x = x_ref[...]
    y = y_ref[...]
    z = z_ref[...]
    cut = cut_ref[...]
    inv = 1.0 / (jnp.sqrt(x * x + y * y + z * z) + 1e-9)
    ux = x * inv
    uy = y * inv
    uz = z * inv
    s3 = float(np.sqrt(3.0))
    s15 = float(np.sqrt(15.0))
    s5h = float(np.sqrt(5.0) / 2.0)
    sh = jnp.concatenate(
        [
            s3 * ux, s3 * uy, s3 * uz,
            s15 * ux * uy, s15 * uy * uz,
            s5h * (3.0 * uz * uz - 1.0),
            s15 * ux * uz,
            (s15 / 2.0) * (ux * ux - uy * uy),
        ],
        axis=1,
    )
    sh_ref[...] = sh
    csh_ref[...] = sh * cut


def _geom_call(x, y, z, cut):
    grid = (_E // _BE,)
    return pl.pallas_call(
        _geom_body,
        grid=grid,
        in_specs=[_rows_spec(_BE, 1)] * 4,
        out_specs=[_rows_spec(_BE, 8), _rows_spec(_BE, 8)],
        out_shape=[
            jax.ShapeDtypeStruct((_E, 8), _F32),
            jax.ShapeDtypeStruct((_E, 8), _F32),
        ],
    )(x, y, z, cut)


def _embed_body(sp_ref, emb_ref, out_ref):
    sp = sp_ref[...]
    oh = (sp == lax.broadcasted_iota(jnp.int32, (sp.shape[0], _NSPEC), 1))
    out_ref[...] = _dot(oh.astype(_F32), emb_ref[...])


def _embed_call(sp, emb):
    grid = (_N // _BN,)
    return pl.pallas_call(
        _embed_body,
        grid=grid,
        in_specs=[_rows_spec(_BN, 1), _full_spec((_NSPEC, _F))],
        out_specs=_rows_spec(_BN, _F),
        out_shape=jax.ShapeDtypeStruct((_N, _F), _F32),
    )(sp, emb)


def _proj_body(scale, nf_ref, ca_ref, cb_ref, w_ref, b_ref,
               r_ref, s_ref, chi_ref):
    nf = nf_ref[...]
    chi = (ca_ref[...] + cb_ref[...]) * scale
    proj = _dot(nf, w_ref[...]) + b_ref[...]
    q = proj[:, 0:128]
    k = proj[:, 128:256]
    v = proj[:, 256:384]
    qg = proj[:, 384:512]
    kg = proj[:, 512:640]
    pad = jnp.zeros((nf.shape[0], 120), _F32)
    r_ref[...] = jnp.concatenate([q, qg, chi, pad], axis=1)
    s_ref[...] = jnp.concatenate([k, v, kg, chi, pad], axis=1)
    chi_ref[...] = chi


def _proj_call(scale, nf, chi_parts, wcat, bcat):
    grid = (_N // _BN,)
    return pl.pallas_call(
        functools.partial(_proj_body, scale),
        grid=grid,
        in_specs=[
            _rows_spec(_BN, _F),
            _rows_spec(_BN, 8),
            _rows_spec(_BN, 8),
            _full_spec((_F, 5 * _F)),
            _full_spec((1, 5 * _F)),
        ],
        out_specs=[
            _rows_spec(_BN, _DR),
            _rows_spec(_BN, _DS),
            _rows_spec(_BN, 8),
        ],
        out_shape=[
            jax.ShapeDtypeStruct((_N, _DR), _F32),
            jax.ShapeDtypeStruct((_N, _DS), _F32),
            jax.ShapeDtypeStruct((_N, 8), _F32),
        ],
    )(nf, *chi_parts, wcat, bcat)


def _edge_body(d_ref, cut_ref, sh_ref, q_ref, qg_ref, chir_ref,
               k_ref, v_ref, kg_ref, chis_ref,
               w1r_ref, b1r_ref, w2r_ref, b2r_ref,
               w1s_ref, b1s_ref, w2s_ref, b2s_ref,
               g1r_ref, gb1r_ref, g2r_ref, gb2r_ref,
               g1s_ref, gb1s_ref, g2s_ref, gb2s_ref,
               msg_ref, msgc_ref):
    d = d_ref[...]
    cut = cut_ref[...]
    sh = sh_ref[...]
    q = q_ref[...]
    qg = qg_ref[...]
    chir = chir_ref[:, 0:8]
    k = k_ref[...]
    v = v_ref[...]
    kg = kg_ref[...]
    chis = chis_ref[:, 0:8]

    mu0 = float(np.exp(-_CUT))
    mu = mu0 + lax.broadcasted_iota(jnp.int32, (1, _K), 1).astype(_F32) * (
        (1.0 - mu0) / (_K - 1))
    beta = float((2.0 / _K * (1.0 - np.exp(-_CUT))) ** (-2))
    t = jnp.exp(-d) - mu
    rbf = jnp.exp(-beta * t * t)

    chi_ij = chis - chir
    chi_sc = _dot(chi_ij * chi_ij, _deg_mask())

    def mlp2(x, w1, b1, w2, b2):
        h = _silu(_dot(x, w1) + b1)
        return _dot(h, w2) + b2

    w = mlp2(rbf, w1r_ref[...], b1r_ref[...], w2r_ref[...], b2r_ref[...]) + \
        mlp2(chi_sc, w1s_ref[...], b1s_ref[...], w2s_ref[...], b2s_ref[...])
    wg = mlp2(rbf, g1r_ref[...], gb1r_ref[...], g2r_ref[...], gb2r_ref[...]) + \
        mlp2(chi_sc, g1s_ref[...], gb1s_ref[...], g2s_ref[...], gb2s_ref[...])

    hm, hmT = _head_masks()
    alpha = _dot(q * w * k, hm) * (cut * (1.0 / float(np.sqrt(_DH))))
    ag = jnp.sum(qg * kg, axis=1, keepdims=True) * (1.0 / float(np.sqrt(_F)))
    coeff = wg * (ag * cut)
    msg_ref[...] = _dot(alpha, hmT) * v
    msgc_ref[...] = _dot(coeff, _rep_mask()) * sh


def _col_spec(nrows, ncols, col0):
    return pl.BlockSpec((nrows, ncols), lambda i, _c=col0: (i, _c))


def _edge_call(d, cut, sh, gr, gs, wts, ne):
    grid = (ne // _BE,)
    w_specs = [_full_spec(w.shape) for w in wts]
    return pl.pallas_call(
        _edge_body,
        grid=grid,
        in_specs=[
            _rows_spec(_BE, 1),
            _rows_spec(_BE, 1),
            _rows_spec(_BE, 8),
            _col_spec(_BE, 128, 0),
            _col_spec(_BE, 128, 1),
            _col_spec(_BE, 128, 2),
            _col_spec(_BE, 128, 0),
            _col_spec(_BE, 128, 1),
            _col_spec(_BE, 128, 2),
            _col_spec(_BE, 128, 3),
        ] + w_specs,
        out_specs=[_rows_spec(_BE, _F), _rows_spec(_BE, 8)],
        out_shape=[
            jax.ShapeDtypeStruct((ne, _F), _F32),
            jax.ShapeDtypeStruct((ne, 8), _F32),
        ],
    )(d, cut, sh, gr, gr, gr, gs, gs, gs, gs, *wts)


def _update_body(nf_ref, chi_ref, a0_ref, a1_ref, c0_ref, c1_ref,
                 w_ref, b_ref, nfo_ref, chio_ref):
    nf1 = nf_ref[...] + (a0_ref[...] + a1_ref[...]) * (1.0 / _AVG)
    chi1 = chi_ref[...] + (c0_ref[...] + c1_ref[...]) * (1.0 / _AVG)
    dn = _dot(chi1 * chi1, _deg_mask())
    feat = jnp.concatenate([nf1, dn], axis=1)
    o = _silu(_dot(feat, w_ref[...]) + b_ref[...])
    nfo_ref[...] = nf1 + o[:, 0:128]
    g = _dot(o[:, 128:130], _rep_mask())
    chio_ref[...] = chi1 + chi1 * g


def _update_call(nf, chi, a_parts, c_parts, wi, bi):
    grid = (_N // _BN,)
    return pl.pallas_call(
        _update_body,
        grid=grid,
        in_specs=[
            _rows_spec(_BN, _F),
            _rows_spec(_BN, 8),
            _rows_spec(_BN, _F),
            _rows_spec(_BN, _F),
            _rows_spec(_BN, 8),
            _rows_spec(_BN, 8),
            _full_spec((_F + 2, _F + 2)),
            _full_spec((1, _F + 2)),
        ],
        out_specs=[_rows_spec(_BN, _F), _rows_spec(_BN, 8)],
        out_shape=[
            jax.ShapeDtypeStruct((_N, _F), _F32),
            jax.ShapeDtypeStruct((_N, 8), _F32),
        ],
    )(nf, chi, *a_parts, *c_parts, wi, bi)


def _head_body(nf_ref, w1_ref, b1_ref, w2_ref, b2_ref, out_ref):
    h = _silu(_dot(nf_ref[...], w1_ref[...]) + b1_ref[...])
    out_ref[...] = _dot(h, w2_ref[...]) + b2_ref[...]


def _head_call(nf, w1, b1, w2, b2):
    grid = (_N // _BN,)
    return pl.pallas_call(
        _head_body,
        grid=grid,
        in_specs=[
            _rows_spec(_BN, _F),
            _full_spec((_F, _F)),
            _full_spec((1, _F)),
            _full_spec((_F, 1)),
            _full_spec((1, 1)),
        ],
        out_specs=_rows_spec(_BN, 1),
        out_shape=jax.ShapeDtypeStruct((_N, 1), _F32),
    )(nf, w1, b1, w2, b2)


def kernel(edge_vectors, distances, cutoffs, node_species, senders, receivers, params):
    x = edge_vectors[:, 0:1]
    y = edge_vectors[:, 1:2]
    z = edge_vectors[:, 2:3]
    d = distances.reshape(_E, 1)
    cut = cutoffs.reshape(_E, 1)
    sp = node_species.reshape(_N, 1).astype(jnp.int32)
    snd = senders.astype(jnp.int32)
    rcv = receivers.astype(jnp.int32)

    sh, csh = _geom_call(x, y, z, cut)
    p = _scatter_rows(csh, rcv)
    chi_parts = [p[0, :_N], p[1, :_N]]
    nf = _embed_call(sp, params['embed'])

    zeros8 = jnp.zeros((_N, 8), _F32)
    scale = 1.0 / _SPHC

    for lp in params['layers']:
        wcat = jnp.concatenate(
            [lp['fb_q'][0], lp['fb_k'][0], lp['fb_v'][0], lp['gb_q'][0], lp['gb_k'][0]],
            axis=1,
        )
        bcat = jnp.concatenate(
            [lp['fb_q'][1], lp['fb_k'][1], lp['fb_v'][1], lp['gb_q'][1], lp['gb_k'][1]],
        ).reshape(1, 5 * _F)
        r_tab, s_tab, chi = _proj_call(scale, nf, chi_parts, wcat, bcat)
        wts = [
            lp['fb_rad'][0][0], lp['fb_rad'][0][1].reshape(1, -1),
            lp['fb_rad'][1][0], lp['fb_rad'][1][1].reshape(1, -1),
            lp['fb_sph'][0][0], lp['fb_sph'][0][1].reshape(1, -1),
            lp['fb_sph'][1][0], lp['fb_sph'][1][1].reshape(1, -1),
            lp['gb_rad'][0][0], lp['gb_rad'][0][1].reshape(1, -1),
            lp['gb_rad'][1][0], lp['gb_rad'][1][1].reshape(1, -1),
            lp['gb_sph'][0][0], lp['gb_sph'][0][1].reshape(1, -1),
            lp['gb_sph'][1][0], lp['gb_sph'][1][1].reshape(1, -1),
        ]
        g_r = _gather_rows(r_tab, rcv)
        g_s = _gather_rows(s_tab, snd)
        msg_nf, msg_chi = _edge_call(d, cut, sh, g_r, g_s, wts, _E)
        acc = _scatter_rows(msg_nf, rcv)
        accc = _scatter_rows(msg_chi, rcv)
        nf, chi_next = _update_call(nf, chi,
                                    [acc[0, :_N], acc[1, :_N]],
                                    [accc[0, :_N], accc[1, :_N]],
                                    lp['inter'][0], lp['inter'][1].reshape(1, -1))
        chi_parts, scale = [chi_next, zeros8], 1.0

    out = _head_call(nf, params['out1'][0], params['out1'][1].reshape(1, -1),
                     params['out2'][0], params['out2'][1].reshape(1, -1))
    return out.reshape(_N)


# speedup vs baseline: 1.0595x; 1.0007x over previous
import functools

import numpy as np
import jax
import jax.numpy as jnp
from jax import lax
from jax.experimental import pallas as pl
from jax.experimental.pallas import tpu as pltpu
from jax.experimental.pallas import tpu_sc as plsc

_N = 10000
_E = 320000
_F = 128
_H = 4
_DH = 32
_K = 32
_CUT = 5.0
_NSPEC = 100
_AVG = 32.0
_SPHC = 32.0

_NC = 2
_NS = 16
_NW = _NC * _NS

_NPAD = 10240
_DR = 384
_DS = 512

_BE = 2560
_BN = 2000

_F32 = jnp.float32


def _pick_group(bpw, d, cap=430_000, kord=(5, 4, 2, 1)):
    for k in kord:
        for ch in range(128, 7, -8):
            if bpw % (ch * k) == 0 and k * ch * d * 4 <= cap:
                return ch, k
    raise ValueError((bpw, d))


def _sc_params(d):
    if d % 128 == 0:
        return None
    return pltpu.CompilerParams(use_tc_tiling_on_sc=False)


def _gather_phase(table, idx, out, idx_v, rows_v, sem_g, sem_w,
                  base, bpw, ch, kk):
    g_iters = bpw // (ch * kk)

    def body(g, carry):
        off = base + g * (kk * ch)
        pltpu.sync_copy(idx.at[pl.ds(off, kk * ch)], idx_v)
        for j in range(kk):
            pltpu.async_copy(
                table.at[idx_v.at[pl.ds(j * ch, ch)]], rows_v.at[j], sem_g)
        for j in range(kk):
            pltpu.make_async_copy(
                table.at[idx_v.at[pl.ds(j * ch, ch)]], rows_v.at[j], sem_g
            ).wait()
            pltpu.async_copy(
                rows_v.at[j], out.at[pl.ds(off + j * ch, ch)], sem_w)
        for j in range(kk):
            pltpu.make_async_copy(
                rows_v.at[j], out.at[pl.ds(off + j * ch, ch)], sem_w
            ).wait()
        return carry

    lax.fori_loop(0, g_iters, body, 0)


@functools.lru_cache(maxsize=None)
def _gather_kernel(n_rows, d, b):
    assert b % _NW == 0 and d % 8 == 0
    bpw = b // _NW
    ch, kk = _pick_group(bpw, d)
    mesh = plsc.VectorSubcoreMesh(core_axis_name="c", subcore_axis_name="s")

    @functools.partial(
        pl.kernel,
        out_type=jax.ShapeDtypeStruct((b, d), _F32),
        mesh=mesh,
        compiler_params=_sc_params(d),
        scratch_types=[
            pltpu.VMEM((kk * ch,), jnp.int32),
            pltpu.VMEM((kk, ch, d), _F32),
            pltpu.SemaphoreType.DMA,
            pltpu.SemaphoreType.DMA,
        ],
    )
    def k(table, idx, out, idx_v, rows_v, sem_g, sem_w):
        wid = lax.axis_index("s") * _NC + lax.axis_index("c")
        base = wid * bpw
        _gather_phase(table, idx, out, idx_v, rows_v, sem_g, sem_w,
                      base, bpw, ch, kk)

    return k


@functools.lru_cache(maxsize=None)
def _scatter_kernel(d, b):
    assert b % _NW == 0 and d % 8 == 0
    bpw = b // _NW
    ch, kk = _pick_group(bpw, d, cap=110_000)
    g_iters = bpw // (ch * kk)
    rpt = _NPAD // _NS
    mesh = plsc.VectorSubcoreMesh(core_axis_name="c", subcore_axis_name="s")

    scratch = [pltpu.VMEM((ch,), jnp.int32) for _ in range(kk)]
    scratch += [
        pltpu.VMEM((kk, ch, d), _F32),
        pltpu.VMEM_SHARED((_NPAD, d), _F32),
        pltpu.SemaphoreType.DMA,
        pltpu.SemaphoreType.DMA,
    ]

    @functools.partial(
        pl.kernel,
        out_type=jax.ShapeDtypeStruct((_NC, _NPAD, d), _F32),
        mesh=mesh,
        compiler_params=_sc_params(d),
        scratch_types=scratch,
    )
    def k(msgs, idx, zeros, out, *rest):
        ixs = rest[:kk]
        msg_v, acc, sem_in, sem_sc = rest[kk:]
        c = lax.axis_index("c")
        s = lax.axis_index("s")
        wid = s * _NC + c
        base = wid * bpw
        pltpu.sync_copy(zeros.at[pl.ds(s * rpt, rpt)], acc.at[pl.ds(s * rpt, rpt)])
        plsc.subcore_barrier()

        def body(g, carry):
            off = base + g * (kk * ch)
            for j in range(kk):
                pltpu.async_copy(idx.at[pl.ds(off + j * ch, ch)], ixs[j], sem_in)
                pltpu.async_copy(msgs.at[pl.ds(off + j * ch, ch)], msg_v.at[j], sem_in)
            for j in range(kk):
                pltpu.make_async_copy(
                    idx.at[pl.ds(off + j * ch, ch)], ixs[j], sem_in).wait()
                pltpu.make_async_copy(
                    msgs.at[pl.ds(off + j * ch, ch)], msg_v.at[j], sem_in).wait()
                pltpu.sync_copy(msg_v.at[j], acc.at[ixs[j]], add=True)
            return carry

        lax.fori_loop(0, g_iters, body, 0)
        plsc.subcore_barrier()
        pltpu.sync_copy(acc.at[pl.ds(s * rpt, rpt)], out.at[c, pl.ds(s * rpt, rpt)])

    return k


def _gather_rows(table, idx):
    n_rows, d = table.shape
    return _gather_kernel(n_rows, d, idx.shape[0])(table, idx)


def _scatter_rows(msgs, idx):
    b, d = msgs.shape
    zeros = jnp.zeros((_NPAD, d), _F32)
    return _scatter_kernel(d, b)(msgs, idx, zeros)


def _rows_spec(nrows, ncols):
    return pl.BlockSpec((nrows, ncols), lambda i: (i, 0))


def _full_spec(shape):
    return pl.BlockSpec(shape, lambda i: tuple(0 for _ in shape))


def _silu(x):
    return x * jax.nn.sigmoid(x)


def _deg_mask():
    r = lax.broadcasted_iota(jnp.int32, (8, 2), 0)
    c = lax.broadcasted_iota(jnp.int32, (8, 2), 1)
    deg = (r >= 3).astype(jnp.int32)
    return (deg == c).astype(_F32)


def _rep_mask():
    r = lax.broadcasted_iota(jnp.int32, (2, 8), 0)
    c = lax.broadcasted_iota(jnp.int32, (2, 8), 1)
    deg = (c >= 3).astype(jnp.int32)
    return (r == deg).astype(_F32)


def _head_masks():
    r = lax.broadcasted_iota(jnp.int32, (_F, _H), 0)
    c = lax.broadcasted_iota(jnp.int32, (_F, _H), 1)
    hm = ((r // _DH) == c).astype(_F32)
    r2 = lax.broadcasted_iota(jnp.int32, (_H, _F), 0)
    c2 = lax.broadcasted_iota(jnp.int32, (_H, _F), 1)
    hmT = (r2 == (c2 // _DH)).astype(_F32)
    return hm, hmT


def _dot(a, b):
    return jnp.dot(a, b, preferred_element_type=_F32)


def _geom_body(x_ref, y_ref, z_ref, cut_ref, sh_ref, csh_ref):
    x = x_ref[...]
    y = y_ref[...]
    z = z_ref[...]
    cut = cut_ref[...]
    inv = 1.0 / (jnp.sqrt(x * x + y * y + z * z) + 1e-9)
    ux = x * inv
    uy = y * inv
    uz = z * inv
    s3 = float(np.sqrt(3.0))
    s15 = float(np.sqrt(15.0))
    s5h = float(np.sqrt(5.0) / 2.0)
    sh = jnp.concatenate(
        [
            s3 * ux, s3 * uy, s3 * uz,
            s15 * ux * uy, s15 * uy * uz,
            s5h * (3.0 * uz * uz - 1.0),
            s15 * ux * uz,
            (s15 / 2.0) * (ux * ux - uy * uy),
        ],
        axis=1,
    )
    sh_ref[...] = sh
    csh_ref[...] = sh * cut


def _geom_call(x, y, z, cut):
    grid = (_E // _BE,)
    return pl.pallas_call(
        _geom_body,
        grid=grid,
        in_specs=[_rows_spec(_BE, 1)] * 4,
        out_specs=[_rows_spec(_BE, 8), _rows_spec(_BE, 8)],
        out_shape=[
            jax.ShapeDtypeStruct((_E, 8), _F32),
            jax.ShapeDtypeStruct((_E, 8), _F32),
        ],
    )(x, y, z, cut)


def _embed_body(sp_ref, emb_ref, out_ref):
    sp = sp_ref[...]
    oh = (sp == lax.broadcasted_iota(jnp.int32, (sp.shape[0], _NSPEC), 1))
    out_ref[...] = _dot(oh.astype(_F32), emb_ref[...])


def _embed_call(sp, emb):
    grid = (_N // _BN,)
    return pl.pallas_call(
        _embed_body,
        grid=grid,
        in_specs=[_rows_spec(_BN, 1), _full_spec((_NSPEC, _F))],
        out_specs=_rows_spec(_BN, _F),
        out_shape=jax.ShapeDtypeStruct((_N, _F), _F32),
    )(sp, emb)


def _proj_body(scale, nf_ref, ca_ref, cb_ref, w_ref, b_ref,
               r_ref, s_ref, chi_ref):
    nf = nf_ref[...]
    chi = (ca_ref[...] + cb_ref[...]) * scale
    proj = _dot(nf, w_ref[...]) + b_ref[...]
    q = proj[:, 0:128]
    k = proj[:, 128:256]
    v = proj[:, 256:384]
    qg = proj[:, 384:512]
    kg = proj[:, 512:640]
    pad = jnp.zeros((nf.shape[0], 120), _F32)
    r_ref[...] = jnp.concatenate([q, qg, chi, pad], axis=1)
    s_ref[...] = jnp.concatenate([k, v, kg, chi, pad], axis=1)
    chi_ref[...] = chi


def _proj_call(scale, nf, chi_parts, wcat, bcat):
    grid = (_N // _BN,)
    return pl.pallas_call(
        functools.partial(_proj_body, scale),
        grid=grid,
        in_specs=[
            _rows_spec(_BN, _F),
            _rows_spec(_BN, 8),
            _rows_spec(_BN, 8),
            _full_spec((_F, 5 * _F)),
            _full_spec((1, 5 * _F)),
        ],
        out_specs=[
            _rows_spec(_BN, _DR),
            _rows_spec(_BN, _DS),
            _rows_spec(_BN, 8),
        ],
        out_shape=[
            jax.ShapeDtypeStruct((_N, _DR), _F32),
            jax.ShapeDtypeStruct((_N, _DS), _F32),
            jax.ShapeDtypeStruct((_N, 8), _F32),
        ],
    )(nf, *chi_parts, wcat, bcat)


def _edge_body(d_ref, cut_ref, sh_ref, q_ref, qg_ref, chir_ref,
               k_ref, v_ref, kg_ref, chis_ref,
               w1r_ref, b1r_ref, w2r_ref, b2r_ref,
               w1s_ref, b1s_ref, w2s_ref, b2s_ref,
               g1r_ref, gb1r_ref, g2r_ref, gb2r_ref,
               g1s_ref, gb1s_ref, g2s_ref, gb2s_ref,
               msg_ref, msgc_ref):
    d = d_ref[...]
    cut = cut_ref[...]
    sh = sh_ref[...]
    q = q_ref[...]
    qg = qg_ref[...]
    chir = chir_ref[:, 0:8]
    k = k_ref[...]
    v = v_ref[...]
    kg = kg_ref[...]
    chis = chis_ref[:, 0:8]

    mu0 = float(np.exp(-_CUT))
    mu = mu0 + lax.broadcasted_iota(jnp.int32, (1, _K), 1).astype(_F32) * (
        (1.0 - mu0) / (_K - 1))
    beta = float((2.0 / _K * (1.0 - np.exp(-_CUT))) ** (-2))
    t = jnp.exp(-d) - mu
    rbf = jnp.exp(-beta * t * t)

    chi_ij = chis - chir
    chi_sc = _dot(chi_ij * chi_ij, _deg_mask())

    def mlp2(x, w1, b1, w2, b2):
        h = _silu(_dot(x, w1) + b1)
        return _dot(h, w2) + b2

    w = mlp2(rbf, w1r_ref[...], b1r_ref[...], w2r_ref[...], b2r_ref[...]) + \
        mlp2(chi_sc, w1s_ref[...], b1s_ref[...], w2s_ref[...], b2s_ref[...])
    wg = mlp2(rbf, g1r_ref[...], gb1r_ref[...], g2r_ref[...], gb2r_ref[...]) + \
        mlp2(chi_sc, g1s_ref[...], gb1s_ref[...], g2s_ref[...], gb2s_ref[...])

    hm, hmT = _head_masks()
    alpha = _dot(q * w * k, hm) * (cut * (1.0 / float(np.sqrt(_DH))))
    ag = jnp.sum(qg * kg, axis=1, keepdims=True) * (1.0 / float(np.sqrt(_F)))
    coeff = wg * (ag * cut)
    msg_ref[...] = _dot(alpha, hmT) * v
    msgc_ref[...] = _dot(coeff, _rep_mask()) * sh


def _col_spec(nrows, ncols, col0):
    return pl.BlockSpec((nrows, ncols), lambda i, _c=col0: (i, _c))


def _edge_call(d, cut, sh, gr, gs, wts, ne):
    grid = (ne // _BE,)
    w_specs = [_full_spec(w.shape) for w in wts]
    return pl.pallas_call(
        _edge_body,
        grid=grid,
        in_specs=[
            _rows_spec(_BE, 1),
            _rows_spec(_BE, 1),
            _rows_spec(_BE, 8),
            _col_spec(_BE, 128, 0),
            _col_spec(_BE, 128, 1),
            _col_spec(_BE, 128, 2),
            _col_spec(_BE, 128, 0),
            _col_spec(_BE, 128, 1),
            _col_spec(_BE, 128, 2),
            _col_spec(_BE, 128, 3),
        ] + w_specs,
        out_specs=[_rows_spec(_BE, _F), _rows_spec(_BE, 8)],
        out_shape=[
            jax.ShapeDtypeStruct((ne, _F), _F32),
            jax.ShapeDtypeStruct((ne, 8), _F32),
        ],
    )(d, cut, sh, gr, gr, gr, gs, gs, gs, gs, *wts)


def _update_body(nf_ref, chi_ref, a0_ref, a1_ref, c0_ref, c1_ref,
                 w_ref, b_ref, nfo_ref, chio_ref):
    nf1 = nf_ref[...] + (a0_ref[...] + a1_ref[...]) * (1.0 / _AVG)
    chi1 = chi_ref[...] + (c0_ref[...] + c1_ref[...]) * (1.0 / _AVG)
    dn = _dot(chi1 * chi1, _deg_mask())
    feat = jnp.concatenate([nf1, dn], axis=1)
    o = _silu(_dot(feat, w_ref[...]) + b_ref[...])
    nfo_ref[...] = nf1 + o[:, 0:128]
    g = _dot(o[:, 128:130], _rep_mask())
    chio_ref[...] = chi1 + chi1 * g


def _update_call(nf, chi, a_parts, c_parts, wi, bi):
    grid = (_N // _BN,)
    return pl.pallas_call(
        _update_body,
        grid=grid,
        in_specs=[
            _rows_spec(_BN, _F),
            _rows_spec(_BN, 8),
            _rows_spec(_BN, _F),
            _rows_spec(_BN, _F),
            _rows_spec(_BN, 8),
            _rows_spec(_BN, 8),
            _full_spec((_F + 2, _F + 2)),
            _full_spec((1, _F + 2)),
        ],
        out_specs=[_rows_spec(_BN, _F), _rows_spec(_BN, 8)],
        out_shape=[
            jax.ShapeDtypeStruct((_N, _F), _F32),
            jax.ShapeDtypeStruct((_N, 8), _F32),
        ],
    )(nf, chi, *a_parts, *c_parts, wi, bi)


def _head_body(nf_ref, w1_ref, b1_ref, w2_ref, b2_ref, out_ref):
    h = _silu(_dot(nf_ref[...], w1_ref[...]) + b1_ref[...])
    out_ref[...] = _dot(h, w2_ref[...]) + b2_ref[...]


def _head_call(nf, w1, b1, w2, b2):
    grid = (_N // _BN,)
    return pl.pallas_call(
        _head_body,
        grid=grid,
        in_specs=[
            _rows_spec(_BN, _F),
            _full_spec((_F, _F)),
            _full_spec((1, _F)),
            _full_spec((_F, 1)),
            _full_spec((1, 1)),
        ],
        out_specs=_rows_spec(_BN, 1),
        out_shape=jax.ShapeDtypeStruct((_N, 1), _F32),
    )(nf, w1, b1, w2, b2)


def kernel(edge_vectors, distances, cutoffs, node_species, senders, receivers, params):
    x = edge_vectors[:, 0:1]
    y = edge_vectors[:, 1:2]
    z = edge_vectors[:, 2:3]
    d = distances.reshape(_E, 1)
    cut = cutoffs.reshape(_E, 1)
    sp = node_species.reshape(_N, 1).astype(jnp.int32)
    snd = senders.astype(jnp.int32)
    rcv = receivers.astype(jnp.int32)

    sh, csh = _geom_call(x, y, z, cut)
    p = _scatter_rows(csh, rcv)
    chi_parts = [p[0, :_N], p[1, :_N]]
    nf = _embed_call(sp, params['embed'])

    zeros8 = jnp.zeros((_N, 8), _F32)
    scale = 1.0 / _SPHC

    for lp in params['layers']:
        wcat = jnp.concatenate(
            [lp['fb_q'][0], lp['fb_k'][0], lp['fb_v'][0], lp['gb_q'][0], lp['gb_k'][0]],
            axis=1,
        )
        bcat = jnp.concatenate(
            [lp['fb_q'][1], lp['fb_k'][1], lp['fb_v'][1], lp['gb_q'][1], lp['gb_k'][1]],
        ).reshape(1, 5 * _F)
        r_tab, s_tab, chi = _proj_call(scale, nf, chi_parts, wcat, bcat)
        wts = [
            lp['fb_rad'][0][0], lp['fb_rad'][0][1].reshape(1, -1),
            lp['fb_rad'][1][0], lp['fb_rad'][1][1].reshape(1, -1),
            lp['fb_sph'][0][0], lp['fb_sph'][0][1].reshape(1, -1),
            lp['fb_sph'][1][0], lp['fb_sph'][1][1].reshape(1, -1),
            lp['gb_rad'][0][0], lp['gb_rad'][0][1].reshape(1, -1),
            lp['gb_rad'][1][0], lp['gb_rad'][1][1].reshape(1, -1),
            lp['gb_sph'][0][0], lp['gb_sph'][0][1].reshape(1, -1),
            lp['gb_sph'][1][0], lp['gb_sph'][1][1].reshape(1, -1),
        ]
        g_r = _gather_rows(r_tab, rcv)
        g_s = _gather_rows(s_tab, snd)
        msg_nf, msg_chi = _edge_call(d, cut, sh, g_r, g_s, wts, _E)
        acc = _scatter_rows(msg_nf, rcv)
        accc = _scatter_rows(msg_chi, rcv)
        nf, chi_next = _update_call(nf, chi,
                                    [acc[0, :_N], acc[1, :_N]],
                                    [accc[0, :_N], accc[1, :_N]],
                                    lp['inter'][0], lp['inter'][1].reshape(1, -1))
        chi_parts, scale = [chi_next, zeros8], 1.0

    out = _head_call(nf, params['out1'][0], params['out1'][1].reshape(1, -1),
                     params['out2'][0], params['out2'][1].reshape(1, -1))
    return out.reshape(_N)
